# baseline (speedup 1.0000x reference)
import jax
import jax.numpy as jnp
from jax.experimental import pallas as pl
from jax.experimental.pallas import tpu as pltpu

_N, _Cin, _T, _V, _M = 64, 3, 300, 25, 2
_K = 3
_CH = [64, 128, 256]
_STR = [1, 2, 2]
_NUM_CLASS = 400


def _stats_kernel(xt_ref, s1_ref, s2_ref):
    b = pl.program_id(0)
    xb = xt_ref[0]
    s1 = jnp.sum(xb, axis=0)
    s2 = jnp.sum(xb * xb, axis=0)

    @pl.when(b < _M)
    def _init():
        s1_ref[0] = s1
        s2_ref[0] = s2

    @pl.when(b >= _M)
    def _acc():
        s1_ref[0] = s1_ref[0] + s1
        s2_ref[0] = s2_ref[0] + s2


def _spatial(h3, Ae, Wg2T, bg, C, D, T):
    hs = jnp.swapaxes(h3, 1, 2)
    hf = hs.reshape(T * C, _V)
    hA = jnp.concatenate(
        [jnp.dot(hf, Ae[k], preferred_element_type=jnp.float32)
         .reshape(T, C, _V) for k in range(_K)], axis=1)
    hA = jnp.swapaxes(hA, 1, 2).reshape(T * _V, _K * C)
    z = jnp.dot(hA, Wg2T, preferred_element_type=jnp.float32)
    return jax.nn.relu(z + bg).reshape(T, _V, D)


def _temporal(z3, Wt9T, bt, s, D, T):
    To = T // s
    zp = jnp.concatenate(
        [jnp.zeros((4, _V, D), jnp.float32), z3,
         jnp.zeros((4, _V, D), jnp.float32)], axis=0)
    acc = jnp.zeros((To * _V, D), jnp.float32)
    if s == 1:
        for dt in range(9):
            sl = zp[dt:dt + To].reshape(To * _V, D)
            acc = acc + jnp.dot(sl, Wt9T[dt],
                                preferred_element_type=jnp.float32)
    else:
        zpr = zp.reshape((T + 8) // 2, 2, _V, D)
        for dt in range(9):
            j = dt // 2
            sl = zpr[j:j + To, dt % 2].reshape(To * _V, D)
            acc = acc + jnp.dot(sl, Wt9T[dt],
                                preferred_element_type=jnp.float32)
    return acc + bt


def _trunk_kernel(xt_ref, scale_ref, shift_ref,
                  A_ref, ei0_ref, ei1_ref, ei2_ref,
                  Wg0_ref, bg0_ref, Wg1_ref, bg1_ref, Wg2_ref, bg2_ref,
                  Wt0_ref, bt0_ref, Wt1_ref, bt1_ref, Wt2_ref, bt2_ref,
                  Wr1_ref, br1_ref, Wr2_ref, br2_ref,
                  out_ref):
    h3 = xt_ref[0] * scale_ref[0] + shift_ref[0]

    A = A_ref[...]
    Wg_r = (Wg0_ref, Wg1_ref, Wg2_ref)
    bg_r = (bg0_ref, bg1_ref, bg2_ref)
    ei_r = (ei0_ref, ei1_ref, ei2_ref)
    Wt_r = (Wt0_ref, Wt1_ref, Wt2_ref)
    bt_r = (bt0_ref, bt1_ref, bt2_ref)
    Wr_r = (None, Wr1_ref, Wr2_ref)
    br_r = (None, br1_ref, br2_ref)

    C, T = _Cin, _T
    for i, (D, s) in enumerate(zip(_CH, _STR)):
        Ae = A * ei_r[i][...]
        z3 = _spatial(h3, Ae, Wg_r[i][...], bg_r[i][...], C, D, T)
        t2 = _temporal(z3, Wt_r[i][...], bt_r[i][...], s, D, T)
        To = T // s
        if i == 0:
            h2 = jax.nn.relu(t2)
        else:
            hr = h3.reshape(To, s, _V, C)[:, 0].reshape(To * _V, C)
            res = jnp.dot(hr, Wr_r[i][...],
                          preferred_element_type=jnp.float32) + br_r[i][...]
            h2 = jax.nn.relu(t2 + res)
        h3 = h2.reshape(To, _V, D)
        C, T = D, To

    out_ref[0, 0] = jnp.mean(h3.reshape(T * _V, C), axis=0)


def _head_kernel(p_ref, wc_ref, bc_ref, out_ref):
    feat = (p_ref[:, 0, :] + p_ref[:, 1, :]) * 0.5
    out_ref[...] = jnp.dot(feat, wc_ref[...],
                           preferred_element_type=jnp.float32) + bc_ref[...]


@jax.jit
def kernel(x, params, A):
    B = _N * _M
    xt = jnp.transpose(x, (0, 4, 2, 3, 1)).reshape(B, _T, _V, _Cin)

    s1, s2 = pl.pallas_call(
        _stats_kernel,
        grid=(B,),
        in_specs=[pl.BlockSpec((1, _T, _V, _Cin), lambda b: (b, 0, 0, 0))],
        out_specs=[pl.BlockSpec((1, _V, _Cin), lambda b: (b % _M, 0, 0)),
                   pl.BlockSpec((1, _V, _Cin), lambda b: (b % _M, 0, 0))],
        out_shape=[jax.ShapeDtypeStruct((_M, _V, _Cin), jnp.float32),
                   jax.ShapeDtypeStruct((_M, _V, _Cin), jnp.float32)],
    )(xt)

    cnt = float(_N * _T)
    mu = s1 / cnt
    var = s2 / cnt - mu * mu
    gamma = params['bn_gamma'].reshape(_M, _V, _Cin)
    beta = params['bn_beta'].reshape(_M, _V, _Cin)
    scale = gamma * jax.lax.rsqrt(var + 1e-5)
    shift = beta - mu * scale
    scale = scale.reshape(_M, 1, _V, _Cin)
    shift = shift.reshape(_M, 1, _V, _Cin)

    wgs, bgs, wts, bts = [], [], [], []
    for i, D in enumerate(_CH):
        Cprev = _Cin if i == 0 else _CH[i - 1]
        wgs.append(jnp.transpose(params['Wg%d' % i], (0, 2, 1))
                   .reshape(_K * Cprev, D))
        bgs.append(params['bg%d' % i].reshape(1, D))
        wts.append(jnp.transpose(params['Wt%d' % i][:, :, :, 0], (2, 1, 0)))
        bts.append(params['bt%d' % i].reshape(1, D))
    wr1 = params['Wr1'][:, :, 0, 0].T
    br1 = params['br1'].reshape(1, _CH[1])
    wr2 = params['Wr2'][:, :, 0, 0].T
    br2 = params['br2'].reshape(1, _CH[2])

    full = lambda a: pl.BlockSpec(a.shape, lambda b: (0,) * a.ndim)
    weights = [A, params['ei0'], params['ei1'], params['ei2'],
               wgs[0], bgs[0], wgs[1], bgs[1], wgs[2], bgs[2],
               wts[0], bts[0], wts[1], bts[1], wts[2], bts[2],
               wr1, br1, wr2, br2]

    pooled = pl.pallas_call(
        _trunk_kernel,
        grid=(B,),
        in_specs=[pl.BlockSpec((1, _T, _V, _Cin), lambda b: (b, 0, 0, 0)),
                  pl.BlockSpec((1, 1, _V, _Cin), lambda b: (b % _M, 0, 0, 0)),
                  pl.BlockSpec((1, 1, _V, _Cin), lambda b: (b % _M, 0, 0, 0))]
                 + [full(w) for w in weights],
        out_specs=pl.BlockSpec((1, 1, _CH[-1]), lambda b: (b, 0, 0)),
        out_shape=jax.ShapeDtypeStruct((B, 1, _CH[-1]), jnp.float32),
        compiler_params=pltpu.CompilerParams(
            dimension_semantics=("parallel",),
            vmem_limit_bytes=100 * 1024 * 1024),
    )(xt, scale, shift, *weights)

    pooled = pooled.reshape(_N, _M, _CH[-1])

    logits = pl.pallas_call(
        _head_kernel,
        in_specs=[pl.BlockSpec(pooled.shape, lambda: (0, 0, 0)),
                  pl.BlockSpec((_CH[-1], _NUM_CLASS), lambda: (0, 0)),
                  pl.BlockSpec((1, _NUM_CLASS), lambda: (0, 0))],
        out_specs=pl.BlockSpec((_N, _NUM_CLASS), lambda: (0, 0)),
        out_shape=jax.ShapeDtypeStruct((_N, _NUM_CLASS), jnp.float32),
    )(pooled, params['Wc'].T, params['bc'].reshape(1, _NUM_CLASS))

    return logits


# speedup vs baseline: 1.1445x; 1.1445x over previous
import jax
import jax.numpy as jnp
from jax.experimental import pallas as pl
from jax.experimental.pallas import tpu as pltpu

_N, _Cin, _T, _V, _M = 64, 3, 300, 25, 2
_K = 3
_CH = [64, 128, 256]
_STR = [1, 2, 2]
_NUM_CLASS = 400


def _stats_kernel(xt_ref, s1_ref, s2_ref):
    b = pl.program_id(0)
    xb = xt_ref[0]
    s1 = jnp.sum(xb, axis=0)
    s2 = jnp.sum(xb * xb, axis=0)

    @pl.when(b < _M)
    def _init():
        s1_ref[0] = s1
        s2_ref[0] = s2

    @pl.when(b >= _M)
    def _acc():
        s1_ref[0] = s1_ref[0] + s1
        s2_ref[0] = s2_ref[0] + s2


def _spatial(h3, Ae, Wg2T, bg, C, D, T):
    hs = jnp.swapaxes(h3, 1, 2)
    hf = hs.reshape(T * C, _V)
    hA = jnp.concatenate(
        [jnp.dot(hf, Ae[k], preferred_element_type=jnp.float32)
         .astype(jnp.bfloat16).reshape(T, C, _V)
         for k in range(_K)], axis=1)
    hA = jnp.swapaxes(hA, 1, 2).reshape(T * _V, _K * C)
    z = jnp.dot(hA, Wg2T, preferred_element_type=jnp.float32)
    return jax.nn.relu(z + bg).astype(jnp.bfloat16).reshape(T, _V, D)


def _temporal(z3, Wt9T, bt, s, D, T):
    To = T // s
    zp = jnp.concatenate(
        [jnp.zeros((4, _V, D), jnp.bfloat16), z3,
         jnp.zeros((4, _V, D), jnp.bfloat16)], axis=0)
    acc = jnp.zeros((To * _V, D), jnp.float32)
    if s == 1:
        for dt in range(9):
            sl = zp[dt:dt + To].reshape(To * _V, D)
            acc = acc + jnp.dot(sl, Wt9T[dt],
                                preferred_element_type=jnp.float32)
    else:
        zpr = zp.reshape((T + 8) // 2, 2, _V, D)
        for dt in range(9):
            j = dt // 2
            sl = zpr[j:j + To, dt % 2].reshape(To * _V, D)
            acc = acc + jnp.dot(sl, Wt9T[dt],
                                preferred_element_type=jnp.float32)
    return acc + bt


def _trunk_kernel(xt_ref, scale_ref, shift_ref,
                  A_ref, ei0_ref, ei1_ref, ei2_ref,
                  Wg0_ref, bg0_ref, Wg1_ref, bg1_ref, Wg2_ref, bg2_ref,
                  Wt0_ref, bt0_ref, Wt1_ref, bt1_ref, Wt2_ref, bt2_ref,
                  Wr1_ref, br1_ref, Wr2_ref, br2_ref,
                  out_ref):
    h3 = (xt_ref[0] * scale_ref[0] + shift_ref[0]).astype(jnp.bfloat16)

    A = A_ref[...]
    Wg_r = (Wg0_ref, Wg1_ref, Wg2_ref)
    bg_r = (bg0_ref, bg1_ref, bg2_ref)
    ei_r = (ei0_ref, ei1_ref, ei2_ref)
    Wt_r = (Wt0_ref, Wt1_ref, Wt2_ref)
    bt_r = (bt0_ref, bt1_ref, bt2_ref)
    Wr_r = (None, Wr1_ref, Wr2_ref)
    br_r = (None, br1_ref, br2_ref)

    C, T = _Cin, _T
    for i, (D, s) in enumerate(zip(_CH, _STR)):
        Ae = A * ei_r[i][...]
        z3 = _spatial(h3, Ae, Wg_r[i][...], bg_r[i][...], C, D, T)
        t2 = _temporal(z3, Wt_r[i][...], bt_r[i][...], s, D, T)
        To = T // s
        if i == 0:
            h2 = jax.nn.relu(t2)
        else:
            hr = h3.reshape(To, s, _V, C)[:, 0].reshape(To * _V, C)
            res = jnp.dot(hr, Wr_r[i][...],
                          preferred_element_type=jnp.float32) + br_r[i][...]
            h2 = jax.nn.relu(t2 + res)
        h3 = h2.astype(jnp.bfloat16).reshape(To, _V, D)
        C, T = D, To

    out_ref[0, 0] = jnp.mean(h2, axis=0)


def _head_kernel(p_ref, wc_ref, bc_ref, out_ref):
    feat = ((p_ref[:, 0, :] + p_ref[:, 1, :]) * 0.5).astype(jnp.bfloat16)
    out_ref[...] = jnp.dot(feat, wc_ref[...],
                           preferred_element_type=jnp.float32) + bc_ref[...]


@jax.jit
def kernel(x, params, A):
    B = _N * _M
    xt = jnp.transpose(x, (0, 4, 2, 3, 1)).reshape(B, _T, _V, _Cin)

    s1, s2 = pl.pallas_call(
        _stats_kernel,
        grid=(B,),
        in_specs=[pl.BlockSpec((1, _T, _V, _Cin), lambda b: (b, 0, 0, 0))],
        out_specs=[pl.BlockSpec((1, _V, _Cin), lambda b: (b % _M, 0, 0)),
                   pl.BlockSpec((1, _V, _Cin), lambda b: (b % _M, 0, 0))],
        out_shape=[jax.ShapeDtypeStruct((_M, _V, _Cin), jnp.float32),
                   jax.ShapeDtypeStruct((_M, _V, _Cin), jnp.float32)],
    )(xt)

    cnt = float(_N * _T)
    mu = s1 / cnt
    var = s2 / cnt - mu * mu
    gamma = params['bn_gamma'].reshape(_M, _V, _Cin)
    beta = params['bn_beta'].reshape(_M, _V, _Cin)
    scale = gamma * jax.lax.rsqrt(var + 1e-5)
    shift = beta - mu * scale
    scale = scale.reshape(_M, 1, _V, _Cin)
    shift = shift.reshape(_M, 1, _V, _Cin)

    bf = jnp.bfloat16
    wgs, bgs, wts, bts = [], [], [], []
    for i, D in enumerate(_CH):
        Cprev = _Cin if i == 0 else _CH[i - 1]
        wgs.append(jnp.transpose(params['Wg%d' % i], (0, 2, 1))
                   .reshape(_K * Cprev, D).astype(bf))
        bgs.append(params['bg%d' % i].reshape(1, D))
        wts.append(jnp.transpose(params['Wt%d' % i][:, :, :, 0],
                                 (2, 1, 0)).astype(bf))
        bts.append(params['bt%d' % i].reshape(1, D))
    wr1 = params['Wr1'][:, :, 0, 0].T.astype(bf)
    br1 = params['br1'].reshape(1, _CH[1])
    wr2 = params['Wr2'][:, :, 0, 0].T.astype(bf)
    br2 = params['br2'].reshape(1, _CH[2])

    full = lambda a: pl.BlockSpec(a.shape, lambda b: (0,) * a.ndim)
    weights = [A.astype(bf), params['ei0'].astype(bf),
               params['ei1'].astype(bf), params['ei2'].astype(bf),
               wgs[0], bgs[0], wgs[1], bgs[1], wgs[2], bgs[2],
               wts[0], bts[0], wts[1], bts[1], wts[2], bts[2],
               wr1, br1, wr2, br2]

    pooled = pl.pallas_call(
        _trunk_kernel,
        grid=(B,),
        in_specs=[pl.BlockSpec((1, _T, _V, _Cin), lambda b: (b, 0, 0, 0)),
                  pl.BlockSpec((1, 1, _V, _Cin), lambda b: (b % _M, 0, 0, 0)),
                  pl.BlockSpec((1, 1, _V, _Cin), lambda b: (b % _M, 0, 0, 0))]
                 + [full(w) for w in weights],
        out_specs=pl.BlockSpec((1, 1, _CH[-1]), lambda b: (b, 0, 0)),
        out_shape=jax.ShapeDtypeStruct((B, 1, _CH[-1]), jnp.float32),
        compiler_params=pltpu.CompilerParams(
            dimension_semantics=("parallel",),
            vmem_limit_bytes=100 * 1024 * 1024),
    )(xt, scale, shift, *weights)

    pooled = pooled.reshape(_N, _M, _CH[-1])

    logits = pl.pallas_call(
        _head_kernel,
        in_specs=[pl.BlockSpec(pooled.shape, lambda: (0, 0, 0)),
                  pl.BlockSpec((_CH[-1], _NUM_CLASS), lambda: (0, 0)),
                  pl.BlockSpec((1, _NUM_CLASS), lambda: (0, 0))],
        out_specs=pl.BlockSpec((_N, _NUM_CLASS), lambda: (0, 0)),
        out_shape=jax.ShapeDtypeStruct((_N, _NUM_CLASS), jnp.float32),
    )(pooled, params['Wc'].T.astype(jnp.bfloat16),
      params['bc'].reshape(1, _NUM_CLASS))

    return logits


# speedup vs baseline: 2.1284x; 1.8597x over previous
import jax
import jax.numpy as jnp
from jax.experimental import pallas as pl
from jax.experimental.pallas import tpu as pltpu

_N, _Cin, _T, _V, _M = 64, 3, 300, 25, 2
_K = 3
_CH = [64, 128, 256]
_STR = [1, 2, 2]
_NUM_CLASS = 400
_VM = _V * _M


def _stats_kernel(x_ref, s1_ref, s2_ref):
    n = pl.program_id(0)
    xb = x_ref[0]
    s1 = jnp.sum(xb, axis=1)
    s2 = jnp.sum(xb * xb, axis=1)

    @pl.when(n == 0)
    def _init():
        s1_ref[...] = s1
        s2_ref[...] = s2

    @pl.when(n > 0)
    def _acc():
        s1_ref[...] = s1_ref[...] + s1
        s2_ref[...] = s2_ref[...] + s2


def _spatial(hs, Ae2, Wg2T, bg, C, D, T):
    hf = hs.reshape(T * C, _VM)
    hA = jnp.concatenate(
        [jnp.dot(hf, Ae2[k], preferred_element_type=jnp.float32)
         .astype(jnp.bfloat16).reshape(T, C, _VM)
         for k in range(_K)], axis=1)
    hA = jnp.swapaxes(hA, 1, 2).reshape(T * _VM, _K * C)
    z = jnp.dot(hA, Wg2T, preferred_element_type=jnp.float32)
    return jax.nn.relu(z + bg).astype(jnp.bfloat16).reshape(T, _VM, D)


def _temporal(z3, Wt9T, bt, s, D, T):
    To = T // s
    zp = jnp.concatenate(
        [jnp.zeros((4, _VM, D), jnp.bfloat16), z3,
         jnp.zeros((4, _VM, D), jnp.bfloat16)], axis=0)
    acc = jnp.zeros((To * _VM, D), jnp.float32)
    if s == 1:
        for dt in range(9):
            sl = zp[dt:dt + To].reshape(To * _VM, D)
            acc = acc + jnp.dot(sl, Wt9T[dt],
                                preferred_element_type=jnp.float32)
    else:
        zpr = zp.reshape((T + 8) // 2, 2, _VM, D)
        for dt in range(9):
            j = dt // 2
            sl = zpr[j:j + To, dt % 2].reshape(To * _VM, D)
            acc = acc + jnp.dot(sl, Wt9T[dt],
                                preferred_element_type=jnp.float32)
    return acc + bt


def _trunk_kernel(x_ref, scale_ref, shift_ref,
                  Ae0_ref, Ae1_ref, Ae2_ref,
                  Wg0_ref, bg0_ref, Wg1_ref, bg1_ref, Wg2_ref, bg2_ref,
                  Wt0_ref, bt0_ref, Wt1_ref, bt1_ref, Wt2_ref, bt2_ref,
                  Wr1_ref, br1_ref, Wr2_ref, br2_ref,
                  out_ref):
    hs = ((x_ref[0] * scale_ref[...]) + shift_ref[...]).astype(jnp.bfloat16)

    Ae_r = (Ae0_ref, Ae1_ref, Ae2_ref)
    Wg_r = (Wg0_ref, Wg1_ref, Wg2_ref)
    bg_r = (bg0_ref, bg1_ref, bg2_ref)
    Wt_r = (Wt0_ref, Wt1_ref, Wt2_ref)
    bt_r = (bt0_ref, bt1_ref, bt2_ref)
    Wr_r = (None, Wr1_ref, Wr2_ref)
    br_r = (None, br1_ref, br2_ref)

    C, T = _Cin, _T
    h3 = None
    for i, (D, s) in enumerate(zip(_CH, _STR)):
        if i == 0:
            hf = hs.reshape(C * T, _VM)
            Ae2 = Ae_r[0][...]
            hA = jnp.concatenate(
                [jnp.dot(hf, Ae2[k_], preferred_element_type=jnp.float32)
                 .astype(jnp.bfloat16).reshape(C, T, _VM)
                 for k_ in range(_K)], axis=0)
            hA = jnp.transpose(hA, (1, 0, 2))
            hA = jnp.swapaxes(hA, 1, 2).reshape(T * _VM, _K * C)
            z = jnp.dot(hA, Wg_r[i][...],
                        preferred_element_type=jnp.float32)
            z3 = jax.nn.relu(z + bg_r[i][...]).astype(jnp.bfloat16)
            z3 = z3.reshape(T, _VM, D)
        else:
            hsw = jnp.swapaxes(h3, 1, 2)
            z3 = _spatial(hsw, Ae_r[i][...], Wg_r[i][...], bg_r[i][...],
                          C, D, T)
        t2 = _temporal(z3, Wt_r[i][...], bt_r[i][...], s, D, T)
        To = T // s
        if i == 0:
            h2 = jax.nn.relu(t2)
        else:
            hr = h3.reshape(To, s, _VM, C)[:, 0].reshape(To * _VM, C)
            res = jnp.dot(hr, Wr_r[i][...],
                          preferred_element_type=jnp.float32) + br_r[i][...]
            h2 = jax.nn.relu(t2 + res)
        h3 = h2.astype(jnp.bfloat16).reshape(To, _VM, D)
        C, T = D, To

    rows = T * _VM
    m_idx = jax.lax.broadcasted_iota(jnp.int32, (rows, 1), 0) % _M
    denom = 1.0 / (T * _V)
    p0 = jnp.sum(jnp.where(m_idx == 0, h2, 0.0), axis=0) * denom
    p1 = jnp.sum(jnp.where(m_idx == 1, h2, 0.0), axis=0) * denom
    out_ref[0, 0] = p0
    out_ref[0, 1] = p1


def _head_kernel(p_ref, wc_ref, bc_ref, out_ref):
    feat = ((p_ref[:, 0, :] + p_ref[:, 1, :]) * 0.5).astype(jnp.bfloat16)
    out_ref[...] = jnp.dot(feat, wc_ref[...],
                           preferred_element_type=jnp.float32) + bc_ref[...]


@jax.jit
def kernel(x, params, A):
    bf = jnp.bfloat16
    xr = x.reshape(_N, _Cin, _T, _VM)

    s1, s2 = pl.pallas_call(
        _stats_kernel,
        grid=(_N,),
        in_specs=[pl.BlockSpec((1, _Cin, _T, _VM), lambda n: (n, 0, 0, 0))],
        out_specs=[pl.BlockSpec((_Cin, _VM), lambda n: (0, 0)),
                   pl.BlockSpec((_Cin, _VM), lambda n: (0, 0))],
        out_shape=[jax.ShapeDtypeStruct((_Cin, _VM), jnp.float32),
                   jax.ShapeDtypeStruct((_Cin, _VM), jnp.float32)],
    )(xr)

    cnt = float(_N * _T)
    mu = s1 / cnt
    var = s2 / cnt - mu * mu
    gamma = jnp.transpose(params['bn_gamma'].reshape(_M, _V, _Cin),
                          (2, 1, 0)).reshape(_Cin, _VM)
    beta = jnp.transpose(params['bn_beta'].reshape(_M, _V, _Cin),
                         (2, 1, 0)).reshape(_Cin, _VM)
    scale = gamma * jax.lax.rsqrt(var + 1e-5)
    shift = beta - mu * scale
    scale = scale.reshape(_Cin, 1, _VM)
    shift = shift.reshape(_Cin, 1, _VM)

    eye2 = jnp.eye(_M, dtype=jnp.float32)
    ae_list = []
    for i in range(3):
        Ae_i = A * params['ei%d' % i]
        Ae2_i = (Ae_i[:, :, None, :, None] *
                 eye2[None, :, None, :]).reshape(_K, _VM, _VM)
        ae_list.append(Ae2_i.astype(bf))

    wgs, bgs, wts, bts = [], [], [], []
    for i, D in enumerate(_CH):
        Cprev = _Cin if i == 0 else _CH[i - 1]
        wgs.append(jnp.transpose(params['Wg%d' % i], (0, 2, 1))
                   .reshape(_K * Cprev, D).astype(bf))
        bgs.append(params['bg%d' % i].reshape(1, D))
        wts.append(jnp.transpose(params['Wt%d' % i][:, :, :, 0],
                                 (2, 1, 0)).astype(bf))
        bts.append(params['bt%d' % i].reshape(1, D))
    wr1 = params['Wr1'][:, :, 0, 0].T.astype(bf)
    br1 = params['br1'].reshape(1, _CH[1])
    wr2 = params['Wr2'][:, :, 0, 0].T.astype(bf)
    br2 = params['br2'].reshape(1, _CH[2])

    full = lambda a: pl.BlockSpec(a.shape, lambda n: (0,) * a.ndim)
    weights = [ae_list[0], ae_list[1], ae_list[2],
               wgs[0], bgs[0], wgs[1], bgs[1], wgs[2], bgs[2],
               wts[0], bts[0], wts[1], bts[1], wts[2], bts[2],
               wr1, br1, wr2, br2]

    pooled = pl.pallas_call(
        _trunk_kernel,
        grid=(_N,),
        in_specs=[pl.BlockSpec((1, _Cin, _T, _VM), lambda n: (n, 0, 0, 0)),
                  full(scale), full(shift)]
                 + [full(w) for w in weights],
        out_specs=pl.BlockSpec((1, _M, _CH[-1]), lambda n: (n, 0, 0)),
        out_shape=jax.ShapeDtypeStruct((_N, _M, _CH[-1]), jnp.float32),
        compiler_params=pltpu.CompilerParams(
            dimension_semantics=("arbitrary",),
            vmem_limit_bytes=100 * 1024 * 1024),
    )(xr, scale, shift, *weights)

    logits = pl.pallas_call(
        _head_kernel,
        in_specs=[pl.BlockSpec(pooled.shape, lambda: (0, 0, 0)),
                  pl.BlockSpec((_CH[-1], _NUM_CLASS), lambda: (0, 0)),
                  pl.BlockSpec((1, _NUM_CLASS), lambda: (0, 0))],
        out_specs=pl.BlockSpec((_N, _NUM_CLASS), lambda: (0, 0)),
        out_shape=jax.ShapeDtypeStruct((_N, _NUM_CLASS), jnp.float32),
    )(pooled, params['Wc'].T.astype(bf),
      params['bc'].reshape(1, _NUM_CLASS))

    return logits


# speedup vs baseline: 2.3064x; 1.0836x over previous
import jax
import jax.numpy as jnp
from jax.experimental import pallas as pl
from jax.experimental.pallas import tpu as pltpu

_N, _Cin, _T, _V, _M = 64, 3, 300, 25, 2
_K = 3
_CH = [64, 128, 256]
_STR = [1, 2, 2]
_NUM_CLASS = 400
_VM = _V * _M


def _stats_kernel(x_ref, s1_ref, s2_ref):
    n = pl.program_id(0)
    xb = x_ref[0]
    s1 = jnp.sum(xb, axis=1)
    s2 = jnp.sum(xb * xb, axis=1)

    @pl.when(n == 0)
    def _init():
        s1_ref[...] = s1
        s2_ref[...] = s2

    @pl.when(n > 0)
    def _acc():
        s1_ref[...] = s1_ref[...] + s1
        s2_ref[...] = s2_ref[...] + s2


def _spatial(hs, Ae2, Wg2T, bg, C, D, T):
    hf = hs.reshape(T * C, _VM)
    hA = jnp.concatenate(
        [jnp.dot(hf, Ae2[k], preferred_element_type=jnp.float32)
         .astype(jnp.bfloat16).reshape(T, C, _VM)
         for k in range(_K)], axis=1)
    hA = jnp.swapaxes(hA, 1, 2).reshape(T * _VM, _K * C)
    z = jnp.dot(hA, Wg2T, preferred_element_type=jnp.float32)
    return jax.nn.relu(z + bg).astype(jnp.bfloat16).reshape(T, _VM, D)


def _temporal(z3, Wt9T, bt, s, D, T):
    To = T // s
    zp = jnp.concatenate(
        [jnp.zeros((4, _VM, D), jnp.bfloat16), z3,
         jnp.zeros((4, _VM, D), jnp.bfloat16)], axis=0)
    if s == 1:
        accs = [jnp.zeros((To * _VM, D), jnp.float32) for _ in range(3)]
        for dt in range(9):
            sl = zp[dt:dt + To].reshape(To * _VM, D)
            accs[dt % 3] = accs[dt % 3] + jnp.dot(
                sl, Wt9T[dt], preferred_element_type=jnp.float32)
        acc = (accs[0] + accs[1]) + (accs[2] + bt)
        return acc
    zpr = zp.reshape((T + 8) // 2, 2, _VM, D)
    win = jnp.concatenate(
        [zpr[dt // 2:dt // 2 + To, dt % 2].reshape(To * _VM, D)
         for dt in range(9)], axis=1)
    wcat = Wt9T.reshape(9 * D, D)
    return jnp.dot(win, wcat,
                   preferred_element_type=jnp.float32) + bt


def _trunk_kernel(x_ref, scale_ref, shift_ref,
                  Ae0_ref, Ae1_ref, Ae2_ref,
                  Wg0_ref, bg0_ref, Wg1_ref, bg1_ref, Wg2_ref, bg2_ref,
                  Wt0_ref, bt0_ref, Wt1_ref, bt1_ref, Wt2_ref, bt2_ref,
                  Wr1_ref, br1_ref, Wr2_ref, br2_ref,
                  out_ref):
    hs = ((x_ref[0] * scale_ref[...]) + shift_ref[...]).astype(jnp.bfloat16)

    Ae_r = (Ae0_ref, Ae1_ref, Ae2_ref)
    Wg_r = (Wg0_ref, Wg1_ref, Wg2_ref)
    bg_r = (bg0_ref, bg1_ref, bg2_ref)
    Wt_r = (Wt0_ref, Wt1_ref, Wt2_ref)
    bt_r = (bt0_ref, bt1_ref, bt2_ref)
    Wr_r = (None, Wr1_ref, Wr2_ref)
    br_r = (None, br1_ref, br2_ref)

    C, T = _Cin, _T
    h3 = None
    for i, (D, s) in enumerate(zip(_CH, _STR)):
        if i == 0:
            hf = hs.reshape(C * T, _VM)
            Ae2 = Ae_r[0][...]
            hA = jnp.concatenate(
                [jnp.dot(hf, Ae2[k_], preferred_element_type=jnp.float32)
                 .astype(jnp.bfloat16).reshape(C, T, _VM)
                 for k_ in range(_K)], axis=0)
            hA = jnp.transpose(hA, (1, 0, 2))
            hA = jnp.swapaxes(hA, 1, 2).reshape(T * _VM, _K * C)
            z = jnp.dot(hA, Wg_r[i][...],
                        preferred_element_type=jnp.float32)
            z3 = jax.nn.relu(z + bg_r[i][...]).astype(jnp.bfloat16)
            z3 = z3.reshape(T, _VM, D)
        else:
            hsw = jnp.swapaxes(h3, 1, 2)
            z3 = _spatial(hsw, Ae_r[i][...], Wg_r[i][...], bg_r[i][...],
                          C, D, T)
        t2 = _temporal(z3, Wt_r[i][...], bt_r[i][...], s, D, T)
        To = T // s
        if i == 0:
            h2 = jax.nn.relu(t2)
        else:
            hr = h3.reshape(To, s, _VM, C)[:, 0].reshape(To * _VM, C)
            res = jnp.dot(hr, Wr_r[i][...],
                          preferred_element_type=jnp.float32) + br_r[i][...]
            h2 = jax.nn.relu(t2 + res)
        h3 = h2.astype(jnp.bfloat16).reshape(To, _VM, D)
        C, T = D, To

    rows = T * _VM
    m_idx = jax.lax.broadcasted_iota(jnp.int32, (rows, 1), 0) % _M
    denom = 1.0 / (T * _V)
    p0 = jnp.sum(jnp.where(m_idx == 0, h2, 0.0), axis=0) * denom
    p1 = jnp.sum(jnp.where(m_idx == 1, h2, 0.0), axis=0) * denom
    out_ref[0, 0] = p0
    out_ref[0, 1] = p1


def _head_kernel(p_ref, wc_ref, bc_ref, out_ref):
    feat = ((p_ref[:, 0, :] + p_ref[:, 1, :]) * 0.5).astype(jnp.bfloat16)
    out_ref[...] = jnp.dot(feat, wc_ref[...],
                           preferred_element_type=jnp.float32) + bc_ref[...]


@jax.jit
def kernel(x, params, A):
    bf = jnp.bfloat16
    xr = x.reshape(_N, _Cin, _T, _VM)

    s1, s2 = pl.pallas_call(
        _stats_kernel,
        grid=(_N,),
        in_specs=[pl.BlockSpec((1, _Cin, _T, _VM), lambda n: (n, 0, 0, 0))],
        out_specs=[pl.BlockSpec((_Cin, _VM), lambda n: (0, 0)),
                   pl.BlockSpec((_Cin, _VM), lambda n: (0, 0))],
        out_shape=[jax.ShapeDtypeStruct((_Cin, _VM), jnp.float32),
                   jax.ShapeDtypeStruct((_Cin, _VM), jnp.float32)],
    )(xr)

    cnt = float(_N * _T)
    mu = s1 / cnt
    var = s2 / cnt - mu * mu
    gamma = jnp.transpose(params['bn_gamma'].reshape(_M, _V, _Cin),
                          (2, 1, 0)).reshape(_Cin, _VM)
    beta = jnp.transpose(params['bn_beta'].reshape(_M, _V, _Cin),
                         (2, 1, 0)).reshape(_Cin, _VM)
    scale = gamma * jax.lax.rsqrt(var + 1e-5)
    shift = beta - mu * scale
    scale = scale.reshape(_Cin, 1, _VM)
    shift = shift.reshape(_Cin, 1, _VM)

    eye2 = jnp.eye(_M, dtype=jnp.float32)
    ae_list = []
    for i in range(3):
        Ae_i = A * params['ei%d' % i]
        Ae2_i = (Ae_i[:, :, None, :, None] *
                 eye2[None, :, None, :]).reshape(_K, _VM, _VM)
        ae_list.append(Ae2_i.astype(bf))

    wgs, bgs, wts, bts = [], [], [], []
    for i, D in enumerate(_CH):
        Cprev = _Cin if i == 0 else _CH[i - 1]
        wgs.append(jnp.transpose(params['Wg%d' % i], (0, 2, 1))
                   .reshape(_K * Cprev, D).astype(bf))
        bgs.append(params['bg%d' % i].reshape(1, D))
        wts.append(jnp.transpose(params['Wt%d' % i][:, :, :, 0],
                                 (2, 1, 0)).astype(bf))
        bts.append(params['bt%d' % i].reshape(1, D))
    wr1 = params['Wr1'][:, :, 0, 0].T.astype(bf)
    br1 = params['br1'].reshape(1, _CH[1])
    wr2 = params['Wr2'][:, :, 0, 0].T.astype(bf)
    br2 = params['br2'].reshape(1, _CH[2])

    full = lambda a: pl.BlockSpec(a.shape, lambda n: (0,) * a.ndim)
    weights = [ae_list[0], ae_list[1], ae_list[2],
               wgs[0], bgs[0], wgs[1], bgs[1], wgs[2], bgs[2],
               wts[0], bts[0], wts[1], bts[1], wts[2], bts[2],
               wr1, br1, wr2, br2]

    pooled = pl.pallas_call(
        _trunk_kernel,
        grid=(_N,),
        in_specs=[pl.BlockSpec((1, _Cin, _T, _VM), lambda n: (n, 0, 0, 0)),
                  full(scale), full(shift)]
                 + [full(w) for w in weights],
        out_specs=pl.BlockSpec((1, _M, _CH[-1]), lambda n: (n, 0, 0)),
        out_shape=jax.ShapeDtypeStruct((_N, _M, _CH[-1]), jnp.float32),
        compiler_params=pltpu.CompilerParams(
            dimension_semantics=("arbitrary",),
            vmem_limit_bytes=100 * 1024 * 1024),
    )(xr, scale, shift, *weights)

    logits = pl.pallas_call(
        _head_kernel,
        in_specs=[pl.BlockSpec(pooled.shape, lambda: (0, 0, 0)),
                  pl.BlockSpec((_CH[-1], _NUM_CLASS), lambda: (0, 0)),
                  pl.BlockSpec((1, _NUM_CLASS), lambda: (0, 0))],
        out_specs=pl.BlockSpec((_N, _NUM_CLASS), lambda: (0, 0)),
        out_shape=jax.ShapeDtypeStruct((_N, _NUM_CLASS), jnp.float32),
    )(pooled, params['Wc'].T.astype(bf),
      params['bc'].reshape(1, _NUM_CLASS))

    return logits
